# baseline (speedup 1.0000x reference)
import functools

import jax
import jax.numpy as jnp
from jax import lax
from jax.experimental import pallas as pl
from jax.experimental.pallas import tpu as pltpu
from jax.experimental.pallas import tpu_sc as plsc

N_NODES = 10000
N_EDGES = 320000
D_FEAT = 128
D_HID = 128
N_LABELS = 64

NC = 2
NS = 16
NW = NC * NS
K = 64
NPAD = 10016
ROWS_PER_TILE = NPAD // NS
SUP = 16
NSUP = 20
C = SUP * NSUP
E_PAD = NS * C * K
DC = 16


@functools.cache
def _make_sc_aggregate(d, with_counts):
  mesh = plsc.VectorSubcoreMesh(
      core_axis_name="c", subcore_axis_name="s", num_cores=NC, num_subcores=NS
  )

  out_type = [jax.ShapeDtypeStruct((NPAD, d), jnp.float32)]
  scratch = [
      pltpu.VMEM((2 * SUP, K), jnp.int32),
      pltpu.VMEM((2 * SUP, K), jnp.int32),
      pltpu.VMEM((K, d), jnp.float32),
      pltpu.VMEM((K, d), jnp.float32),
      pltpu.VMEM((K, d), jnp.float32),
      pltpu.VMEM((K, d), jnp.float32),
      pltpu.VMEM_SHARED((NPAD, d), jnp.float32),
      pltpu.SemaphoreType.DMA,
      pltpu.SemaphoreType.DMA,
      pltpu.SemaphoreType.DMA,
      pltpu.SemaphoreType.DMA,
      pltpu.SemaphoreType.DMA,
      pltpu.SemaphoreType.DMA,
      pltpu.SemaphoreType.DMA,
      pltpu.SemaphoreType.DMA,
      pltpu.SemaphoreType.DMA,
  ]
  if with_counts:
    out_type.append(jax.ShapeDtypeStruct((NPAD, DC), jnp.float32))
    scratch += [
        pltpu.VMEM((K, DC), jnp.float32),
        pltpu.VMEM_SHARED((NPAD, DC), jnp.float32),
        pltpu.SemaphoreType.DMA,
        pltpu.SemaphoreType.DMA,
        pltpu.SemaphoreType.DMA,
        pltpu.SemaphoreType.DMA,
    ]

  @functools.partial(
      pl.kernel,
      out_type=tuple(out_type),
      mesh=mesh,
      scratch_types=tuple(scratch),
      compiler_params=pltpu.CompilerParams(use_tc_tiling_on_sc=False),
  )
  def sc_aggregate(src_h, dst_h, table_h, zeros_h, *rest):
    if with_counts:
      (zeros_c_h, ones_h, out_h, cout_h,
       srcv, dstv, ra0, ra1, rb0, rb1, acc_sh,
       ga0, ga1, gb0, gb1, sa0, sa1, sb0, sb1, isem,
       ones_v, cnt_sh, ca0, ca1, cb0, cb1) = rest
      csem = ((ca0, ca1), (cb0, cb1))
    else:
      (out_h,
       srcv, dstv, ra0, ra1, rb0, rb1, acc_sh,
       ga0, ga1, gb0, gb1, sa0, sa1, sb0, sb1, isem) = rest

    core = lax.axis_index("c")
    s = lax.axis_index("s")
    base = s * C

    rows = ((ra0, ra1), (rb0, rb1))
    gsem = ((ga0, ga1), (gb0, gb1))
    ssem = ((sa0, sa1), (sb0, sb1))

    @pl.when(core == 0)
    def _():
      pltpu.sync_copy(zeros_h,
                      acc_sh.at[pl.ds(s * ROWS_PER_TILE, ROWS_PER_TILE)])
      if with_counts:
        pltpu.sync_copy(
            zeros_c_h, cnt_sh.at[pl.ds(s * ROWS_PER_TILE, ROWS_PER_TILE)])
        pltpu.sync_copy(ones_h, ones_v)

      pltpu.sync_copy(src_h.at[pl.ds(base, SUP)], srcv.at[pl.ds(0, SUP)])
      pltpu.sync_copy(dst_h.at[pl.ds(base, SUP)], dstv.at[pl.ds(0, SUP)])

    plsc.subcore_barrier()

    @pl.when(core == 0)
    def _():
      for bank in range(2):
        for b in range(2):
          i = 2 * bank + b
          pltpu.async_copy(table_h.at[srcv.at[i]], rows[bank][b],
                           gsem[bank][b])

    def quad(t, q):
      for bank in range(2):
        for b in range(2):
          i = t * SUP + q * 4 + 2 * bank + b
          r = lax.rem(i, 2 * SUP)
          pltpu.make_async_copy(
              table_h.at[srcv.at[r]], rows[bank][b], gsem[bank][b]).wait()
          pltpu.async_copy(
              rows[bank][b], acc_sh.at[dstv.at[r]], ssem[bank][b], add=True)
          if with_counts:
            pltpu.async_copy(
                ones_v, cnt_sh.at[dstv.at[r]], csem[bank][b], add=True)
        for b in range(2):
          i = t * SUP + q * 4 + 2 * bank + b
          r = lax.rem(i, 2 * SUP)
          pltpu.make_async_copy(
              rows[bank][b], acc_sh.at[dstv.at[r]], ssem[bank][b]).wait()
          if with_counts:
            pltpu.make_async_copy(
                ones_v, cnt_sh.at[dstv.at[r]], csem[bank][b]).wait()

          @pl.when(i + 4 < C)
          def _():
            rn = lax.rem(i + 4, 2 * SUP)
            pltpu.async_copy(
                table_h.at[srcv.at[rn]], rows[bank][b], gsem[bank][b])

    def superstep(t, carry):
      nxt = lax.rem((t + 1) * SUP, 2 * SUP)

      @pl.when(t + 1 < NSUP)
      def _():
        pltpu.async_copy(src_h.at[pl.ds(base + (t + 1) * SUP, SUP)],
                         srcv.at[pl.ds(nxt, SUP)], isem)
        pltpu.async_copy(dst_h.at[pl.ds(base + (t + 1) * SUP, SUP)],
                         dstv.at[pl.ds(nxt, SUP)], isem)

      def inner(q, carry2):
        quad(t, q)
        return carry2

      lax.fori_loop(0, 3, inner, 0)

      @pl.when(t + 1 < NSUP)
      def _():
        pltpu.make_async_copy(src_h.at[pl.ds(base + (t + 1) * SUP, SUP)],
                              srcv.at[pl.ds(nxt, SUP)], isem).wait()
        pltpu.make_async_copy(dst_h.at[pl.ds(base + (t + 1) * SUP, SUP)],
                              dstv.at[pl.ds(nxt, SUP)], isem).wait()

      quad(t, 3)
      return carry

    @pl.when(core == 0)
    def _():
      lax.fori_loop(0, NSUP, superstep, 0)

    plsc.subcore_barrier()

    @pl.when(core == 0)
    def _():
      pltpu.sync_copy(
          acc_sh.at[pl.ds(s * ROWS_PER_TILE, ROWS_PER_TILE)],
          out_h.at[pl.ds(s * ROWS_PER_TILE, ROWS_PER_TILE)],
      )
      if with_counts:
        pltpu.sync_copy(
            cnt_sh.at[pl.ds(s * ROWS_PER_TILE, ROWS_PER_TILE)],
            cout_h.at[pl.ds(s * ROWS_PER_TILE, ROWS_PER_TILE)],
        )

  return sc_aggregate


BN = 2504


def _layer1_body(parts_ref, cnt_ref, x_ref, wl_ref, wr_ref, b_ref,
                 h_ref, inv_ref):
  cnt = cnt_ref[:, 0:1]
  inv = 1.0 / jnp.maximum(cnt, 1.0)
  agg = parts_ref[...] * inv
  h = jnp.dot(agg, wl_ref[...], preferred_element_type=jnp.float32)
  h += jnp.dot(x_ref[...], wr_ref[...], preferred_element_type=jnp.float32)
  h += b_ref[...]
  h_ref[...] = jnp.maximum(h, 0.0)
  inv_ref[...] = jnp.broadcast_to(inv, (BN, D_HID))


def _layer2_body(parts_ref, h_ref, inv_ref, wl_ref, wr_ref, b_ref, o_ref):
  agg = parts_ref[...] * inv_ref[:, 0:1]
  o = jnp.dot(agg, wl_ref[...], preferred_element_type=jnp.float32)
  o += jnp.dot(h_ref[...], wr_ref[...], preferred_element_type=jnp.float32)
  o_ref[...] = o + b_ref[...]


def _tc_layer1(parts, cnts, x_pad, W1_l, W1_r, b1):
  grid = (NPAD // BN,)
  return pl.pallas_call(
      _layer1_body,
      grid=grid,
      in_specs=[
          pl.BlockSpec((BN, D_FEAT), lambda i: (i, 0)),
          pl.BlockSpec((BN, DC), lambda i: (i, 0)),
          pl.BlockSpec((BN, D_FEAT), lambda i: (i, 0)),
          pl.BlockSpec((D_FEAT, D_HID), lambda i: (0, 0)),
          pl.BlockSpec((D_FEAT, D_HID), lambda i: (0, 0)),
          pl.BlockSpec((1, D_HID), lambda i: (0, 0)),
      ],
      out_specs=[
          pl.BlockSpec((BN, D_HID), lambda i: (i, 0)),
          pl.BlockSpec((BN, D_HID), lambda i: (i, 0)),
      ],
      out_shape=[
          jax.ShapeDtypeStruct((NPAD, D_HID), jnp.float32),
          jax.ShapeDtypeStruct((NPAD, D_HID), jnp.float32),
      ],
  )(parts, cnts, x_pad, W1_l, W1_r, b1)


def _tc_layer2(parts, h, inv_b, W2_l, W2_r, b2):
  grid = (NPAD // BN,)
  return pl.pallas_call(
      _layer2_body,
      grid=grid,
      in_specs=[
          pl.BlockSpec((BN, D_HID), lambda i: (i, 0)),
          pl.BlockSpec((BN, D_HID), lambda i: (i, 0)),
          pl.BlockSpec((BN, D_HID), lambda i: (i, 0)),
          pl.BlockSpec((D_HID, N_LABELS), lambda i: (0, 0)),
          pl.BlockSpec((D_HID, N_LABELS), lambda i: (0, 0)),
          pl.BlockSpec((1, N_LABELS), lambda i: (0, 0)),
      ],
      out_specs=pl.BlockSpec((BN, N_LABELS), lambda i: (i, 0)),
      out_shape=jax.ShapeDtypeStruct((NPAD, N_LABELS), jnp.float32),
  )(parts, h, inv_b, W2_l, W2_r, b2)


def kernel(x, edge_index, W1_l, W1_r, b1, W2_l, W2_r, b2):
  src = edge_index[0].astype(jnp.int32)
  dst = edge_index[1].astype(jnp.int32)
  pad = jnp.full((E_PAD - N_EDGES,), N_NODES, dtype=jnp.int32)
  src_p = jnp.concatenate([src, pad]).reshape(NS * C, K)
  dst_p = jnp.concatenate([dst, pad]).reshape(NS * C, K)

  x_pad = jnp.zeros((NPAD, D_FEAT), jnp.float32).at[:N_NODES].set(x)

  zeros_d = jnp.zeros((ROWS_PER_TILE, D_FEAT), jnp.float32)
  zeros_c = jnp.zeros((ROWS_PER_TILE, DC), jnp.float32)
  ones_b = jnp.ones((K, DC), jnp.float32)

  parts1, cnts = _make_sc_aggregate(D_FEAT, True)(
      src_p, dst_p, x_pad, zeros_d, zeros_c, ones_b)
  h, inv_b = _tc_layer1(parts1, cnts, x_pad, W1_l, W1_r,
                        b1.reshape(1, D_HID))
  (parts2,) = _make_sc_aggregate(D_HID, False)(src_p, dst_p, h, zeros_d)
  out = _tc_layer2(parts2, h, inv_b, W2_l, W2_r, b2.reshape(1, N_LABELS))
  return out[:N_NODES]


# speedup vs baseline: 1.0004x; 1.0004x over previous
import functools

import jax
import jax.numpy as jnp
from jax import lax
from jax.experimental import pallas as pl
from jax.experimental.pallas import tpu as pltpu
from jax.experimental.pallas import tpu_sc as plsc

N_NODES = 10000
N_EDGES = 320000
D_FEAT = 128
D_HID = 128
N_LABELS = 64

NC = 2
NS = 16
NW = NC * NS
K = 64
NPAD = 10016
ROWS_PER_TILE = NPAD // NS
SUP = 16
NSUP = 20
C = SUP * NSUP
E_PAD = NS * C * K
DC = 16


@functools.cache
def _make_sc_aggregate(d, with_counts):
  mesh = plsc.VectorSubcoreMesh(
      core_axis_name="c", subcore_axis_name="s", num_cores=NC, num_subcores=NS
  )

  out_type = [jax.ShapeDtypeStruct((NPAD, d), jnp.float32)]
  scratch = [
      pltpu.VMEM((2 * SUP, K), jnp.int32),
      pltpu.VMEM((2 * SUP, K), jnp.int32),
      pltpu.VMEM((K, d), jnp.float32),
      pltpu.VMEM((K, d), jnp.float32),
      pltpu.VMEM((K, d), jnp.float32),
      pltpu.VMEM((K, d), jnp.float32),
      pltpu.VMEM_SHARED((NPAD, d), jnp.float32),
      pltpu.SemaphoreType.DMA,
      pltpu.SemaphoreType.DMA,
      pltpu.SemaphoreType.DMA,
      pltpu.SemaphoreType.DMA,
      pltpu.SemaphoreType.DMA,
      pltpu.SemaphoreType.DMA,
      pltpu.SemaphoreType.DMA,
      pltpu.SemaphoreType.DMA,
      pltpu.SemaphoreType.DMA,
  ]
  if with_counts:
    out_type.append(jax.ShapeDtypeStruct((NPAD, DC), jnp.float32))
    scratch += [
        pltpu.VMEM((K, DC), jnp.float32),
        pltpu.VMEM_SHARED((NPAD, DC), jnp.float32),
        pltpu.SemaphoreType.DMA,
        pltpu.SemaphoreType.DMA,
        pltpu.SemaphoreType.DMA,
        pltpu.SemaphoreType.DMA,
    ]

  @functools.partial(
      pl.kernel,
      out_type=tuple(out_type),
      mesh=mesh,
      scratch_types=tuple(scratch),
      compiler_params=pltpu.CompilerParams(use_tc_tiling_on_sc=False),
  )
  def sc_aggregate(src_h, dst_h, table_h, zeros_h, *rest):
    if with_counts:
      (zeros_c_h, ones_h, out_h, cout_h,
       srcv, dstv, ra0, ra1, rb0, rb1, acc_sh,
       ga0, ga1, gb0, gb1, sa0, sa1, sb0, sb1, isem,
       ones_v, cnt_sh, ca0, ca1, cb0, cb1) = rest
      csem = ((ca0, ca1), (cb0, cb1))
    else:
      (out_h,
       srcv, dstv, ra0, ra1, rb0, rb1, acc_sh,
       ga0, ga1, gb0, gb1, sa0, sa1, sb0, sb1, isem) = rest

    core = lax.axis_index("c")
    s = lax.axis_index("s")
    base = s * C

    rows = ((ra0, ra1), (rb0, rb1))
    gsem = ((ga0, ga1), (gb0, gb1))
    ssem = ((sa0, sa1), (sb0, sb1))

    @pl.when(core == 0)
    def _():
      with jax.named_scope("zero_stage"):
        pltpu.sync_copy(zeros_h,
                        acc_sh.at[pl.ds(s * ROWS_PER_TILE, ROWS_PER_TILE)])
        if with_counts:
          pltpu.sync_copy(
              zeros_c_h, cnt_sh.at[pl.ds(s * ROWS_PER_TILE, ROWS_PER_TILE)])
          pltpu.sync_copy(ones_h, ones_v)

        pltpu.sync_copy(src_h.at[pl.ds(base, SUP)], srcv.at[pl.ds(0, SUP)])
        pltpu.sync_copy(dst_h.at[pl.ds(base, SUP)], dstv.at[pl.ds(0, SUP)])

    plsc.subcore_barrier()

    @pl.when(core == 0)
    def _():
      for bank in range(2):
        for b in range(2):
          i = 2 * bank + b
          pltpu.async_copy(table_h.at[srcv.at[i]], rows[bank][b],
                           gsem[bank][b])

    def quad(t, q):
      for bank in range(2):
        for b in range(2):
          i = t * SUP + q * 4 + 2 * bank + b
          r = lax.rem(i, 2 * SUP)
          pltpu.make_async_copy(
              table_h.at[srcv.at[r]], rows[bank][b], gsem[bank][b]).wait()
          pltpu.async_copy(
              rows[bank][b], acc_sh.at[dstv.at[r]], ssem[bank][b], add=True)
          if with_counts:
            pltpu.async_copy(
                ones_v, cnt_sh.at[dstv.at[r]], csem[bank][b], add=True)
        for b in range(2):
          i = t * SUP + q * 4 + 2 * bank + b
          r = lax.rem(i, 2 * SUP)
          pltpu.make_async_copy(
              rows[bank][b], acc_sh.at[dstv.at[r]], ssem[bank][b]).wait()
          if with_counts:
            pltpu.make_async_copy(
                ones_v, cnt_sh.at[dstv.at[r]], csem[bank][b]).wait()

          @pl.when(i + 4 < C)
          def _():
            rn = lax.rem(i + 4, 2 * SUP)
            pltpu.async_copy(
                table_h.at[srcv.at[rn]], rows[bank][b], gsem[bank][b])

    def superstep(t, carry):
      nxt = lax.rem((t + 1) * SUP, 2 * SUP)

      @pl.when(t + 1 < NSUP)
      def _():
        pltpu.async_copy(src_h.at[pl.ds(base + (t + 1) * SUP, SUP)],
                         srcv.at[pl.ds(nxt, SUP)], isem)
        pltpu.async_copy(dst_h.at[pl.ds(base + (t + 1) * SUP, SUP)],
                         dstv.at[pl.ds(nxt, SUP)], isem)

      def inner(q, carry2):
        quad(t, q)
        return carry2

      lax.fori_loop(0, 3, inner, 0)

      @pl.when(t + 1 < NSUP)
      def _():
        pltpu.make_async_copy(src_h.at[pl.ds(base + (t + 1) * SUP, SUP)],
                              srcv.at[pl.ds(nxt, SUP)], isem).wait()
        pltpu.make_async_copy(dst_h.at[pl.ds(base + (t + 1) * SUP, SUP)],
                              dstv.at[pl.ds(nxt, SUP)], isem).wait()

      quad(t, 3)
      return carry

    @pl.when(core == 0)
    def _():
      with jax.named_scope("edge_loop"):
        lax.fori_loop(0, NSUP, superstep, 0)

    plsc.subcore_barrier()

    @pl.when(core == 0)
    def _():
      pltpu.sync_copy(
          acc_sh.at[pl.ds(s * ROWS_PER_TILE, ROWS_PER_TILE)],
          out_h.at[pl.ds(s * ROWS_PER_TILE, ROWS_PER_TILE)],
      )
      if with_counts:
        pltpu.sync_copy(
            cnt_sh.at[pl.ds(s * ROWS_PER_TILE, ROWS_PER_TILE)],
            cout_h.at[pl.ds(s * ROWS_PER_TILE, ROWS_PER_TILE)],
        )

  return sc_aggregate


BN = 2504


def _layer1_body(parts_ref, cnt_ref, x_ref, wl_ref, wr_ref, b_ref,
                 h_ref, inv_ref):
  cnt = cnt_ref[:, 0:1]
  inv = 1.0 / jnp.maximum(cnt, 1.0)
  agg = parts_ref[...] * inv
  h = jnp.dot(agg, wl_ref[...], preferred_element_type=jnp.float32)
  h += jnp.dot(x_ref[...], wr_ref[...], preferred_element_type=jnp.float32)
  h += b_ref[...]
  h_ref[...] = jnp.maximum(h, 0.0)
  inv_ref[...] = jnp.broadcast_to(inv, (BN, D_HID))


def _layer2_body(parts_ref, h_ref, inv_ref, wl_ref, wr_ref, b_ref, o_ref):
  agg = parts_ref[...] * inv_ref[:, 0:1]
  o = jnp.dot(agg, wl_ref[...], preferred_element_type=jnp.float32)
  o += jnp.dot(h_ref[...], wr_ref[...], preferred_element_type=jnp.float32)
  o_ref[...] = o + b_ref[...]


def _tc_layer1(parts, cnts, x_pad, W1_l, W1_r, b1):
  grid = (NPAD // BN,)
  return pl.pallas_call(
      _layer1_body,
      grid=grid,
      in_specs=[
          pl.BlockSpec((BN, D_FEAT), lambda i: (i, 0)),
          pl.BlockSpec((BN, DC), lambda i: (i, 0)),
          pl.BlockSpec((BN, D_FEAT), lambda i: (i, 0)),
          pl.BlockSpec((D_FEAT, D_HID), lambda i: (0, 0)),
          pl.BlockSpec((D_FEAT, D_HID), lambda i: (0, 0)),
          pl.BlockSpec((1, D_HID), lambda i: (0, 0)),
      ],
      out_specs=[
          pl.BlockSpec((BN, D_HID), lambda i: (i, 0)),
          pl.BlockSpec((BN, D_HID), lambda i: (i, 0)),
      ],
      out_shape=[
          jax.ShapeDtypeStruct((NPAD, D_HID), jnp.float32),
          jax.ShapeDtypeStruct((NPAD, D_HID), jnp.float32),
      ],
  )(parts, cnts, x_pad, W1_l, W1_r, b1)


def _tc_layer2(parts, h, inv_b, W2_l, W2_r, b2):
  grid = (NPAD // BN,)
  return pl.pallas_call(
      _layer2_body,
      grid=grid,
      in_specs=[
          pl.BlockSpec((BN, D_HID), lambda i: (i, 0)),
          pl.BlockSpec((BN, D_HID), lambda i: (i, 0)),
          pl.BlockSpec((BN, D_HID), lambda i: (i, 0)),
          pl.BlockSpec((D_HID, N_LABELS), lambda i: (0, 0)),
          pl.BlockSpec((D_HID, N_LABELS), lambda i: (0, 0)),
          pl.BlockSpec((1, N_LABELS), lambda i: (0, 0)),
      ],
      out_specs=pl.BlockSpec((BN, N_LABELS), lambda i: (i, 0)),
      out_shape=jax.ShapeDtypeStruct((NPAD, N_LABELS), jnp.float32),
  )(parts, h, inv_b, W2_l, W2_r, b2)


def kernel(x, edge_index, W1_l, W1_r, b1, W2_l, W2_r, b2):
  src = edge_index[0].astype(jnp.int32)
  dst = edge_index[1].astype(jnp.int32)
  pad = jnp.full((E_PAD - N_EDGES,), N_NODES, dtype=jnp.int32)
  src_p = jnp.concatenate([src, pad]).reshape(NS * C, K)
  dst_p = jnp.concatenate([dst, pad]).reshape(NS * C, K)

  x_pad = jnp.zeros((NPAD, D_FEAT), jnp.float32).at[:N_NODES].set(x)

  zeros_d = jnp.zeros((ROWS_PER_TILE, D_FEAT), jnp.float32)
  zeros_c = jnp.zeros((ROWS_PER_TILE, DC), jnp.float32)
  ones_b = jnp.ones((K, DC), jnp.float32)

  parts1, cnts = _make_sc_aggregate(D_FEAT, True)(
      src_p, dst_p, x_pad, zeros_d, zeros_c, ones_b)
  h, inv_b = _tc_layer1(parts1, cnts, x_pad, W1_l, W1_r,
                        b1.reshape(1, D_HID))
  (parts2,) = _make_sc_aggregate(D_HID, False)(src_p, dst_p, h, zeros_d)
  out = _tc_layer2(parts2, h, inv_b, W2_l, W2_r, b2.reshape(1, N_LABELS))
  return out[:N_NODES]


# speedup vs baseline: 2.9111x; 2.9101x over previous
import functools

import jax
import jax.numpy as jnp
from jax import lax
from jax.experimental import pallas as pl
from jax.experimental.pallas import tpu as pltpu
from jax.experimental.pallas import tpu_sc as plsc

N_NODES = 10000
N_EDGES = 320000
D_FEAT = 128
D_HID = 128
N_LABELS = 64

NC = 2
NS = 16
NW = NC * NS
K = 64
NPAD = 10016
ROWS_PER_TILE = NPAD // NS
SUP = 16
NSUP = 10
C = SUP * NSUP
E_PAD = NW * C * K
EPW = C * K
REAL_PW = N_EDGES // NW
PAD_PW = EPW - REAL_PW
DC = 16


@functools.cache
def _make_sc_aggregate(d, with_counts):
  mesh = plsc.VectorSubcoreMesh(
      core_axis_name="c", subcore_axis_name="s", num_cores=NC, num_subcores=NS
  )

  out_type = [jax.ShapeDtypeStruct((NC, NPAD, d), jnp.float32)]
  scratch = [
      pltpu.VMEM((2 * SUP, K), jnp.int32),
      pltpu.VMEM((2 * SUP, K), jnp.int32),
      pltpu.VMEM((K, d), jnp.float32),
      pltpu.VMEM((K, d), jnp.float32),
      pltpu.VMEM((K, d), jnp.float32),
      pltpu.VMEM((K, d), jnp.float32),
      pltpu.VMEM_SHARED((NPAD, d), jnp.float32),
      pltpu.SemaphoreType.DMA,
      pltpu.SemaphoreType.DMA,
      pltpu.SemaphoreType.DMA,
      pltpu.SemaphoreType.DMA,
      pltpu.SemaphoreType.DMA,
      pltpu.SemaphoreType.DMA,
      pltpu.SemaphoreType.DMA,
      pltpu.SemaphoreType.DMA,
      pltpu.SemaphoreType.DMA,
  ]
  if with_counts:
    out_type.append(jax.ShapeDtypeStruct((NC, NPAD, DC), jnp.float32))
    scratch += [
        pltpu.VMEM((K, DC), jnp.float32),
        pltpu.VMEM_SHARED((NPAD, DC), jnp.float32),
        pltpu.SemaphoreType.DMA,
        pltpu.SemaphoreType.DMA,
        pltpu.SemaphoreType.DMA,
        pltpu.SemaphoreType.DMA,
    ]

  @functools.partial(
      pl.kernel,
      out_type=tuple(out_type),
      mesh=mesh,
      scratch_types=tuple(scratch),
      compiler_params=pltpu.CompilerParams(use_tc_tiling_on_sc=False),
  )
  def sc_aggregate(src_h, dst_h, table_h, zeros_h, *rest):
    if with_counts:
      (zeros_c_h, ones_h, out_h, cout_h,
       srcv, dstv, ra0, ra1, rb0, rb1, acc_sh,
       ga0, ga1, gb0, gb1, sa0, sa1, sb0, sb1, isem,
       ones_v, cnt_sh, ca0, ca1, cb0, cb1) = rest
      csem = ((ca0, ca1), (cb0, cb1))
    else:
      (out_h,
       srcv, dstv, ra0, ra1, rb0, rb1, acc_sh,
       ga0, ga1, gb0, gb1, sa0, sa1, sb0, sb1, isem) = rest

    core = lax.axis_index("c")
    s = lax.axis_index("s")
    wid = s * NC + core
    base = wid * C

    rows = ((ra0, ra1), (rb0, rb1))
    gsem = ((ga0, ga1), (gb0, gb1))
    ssem = ((sa0, sa1), (sb0, sb1))

    with jax.named_scope("zero_stage"):
      pltpu.sync_copy(zeros_h,
                      acc_sh.at[pl.ds(s * ROWS_PER_TILE, ROWS_PER_TILE)])
      if with_counts:
        pltpu.sync_copy(
            zeros_c_h, cnt_sh.at[pl.ds(s * ROWS_PER_TILE, ROWS_PER_TILE)])
        pltpu.sync_copy(ones_h, ones_v)

      pltpu.sync_copy(src_h.at[pl.ds(base, SUP)], srcv.at[pl.ds(0, SUP)])
      pltpu.sync_copy(dst_h.at[pl.ds(base, SUP)], dstv.at[pl.ds(0, SUP)])

    plsc.subcore_barrier()

    for bank in range(2):
      for b in range(2):
        i = 2 * bank + b
        pltpu.async_copy(table_h.at[srcv.at[i]], rows[bank][b],
                         gsem[bank][b])

    def quad(t, q):
      for bank in range(2):
        for b in range(2):
          i = t * SUP + q * 4 + 2 * bank + b
          r = lax.rem(i, 2 * SUP)
          pltpu.make_async_copy(
              table_h.at[srcv.at[r]], rows[bank][b], gsem[bank][b]).wait()
          pltpu.async_copy(
              rows[bank][b], acc_sh.at[dstv.at[r]], ssem[bank][b], add=True)
          if with_counts:
            pltpu.async_copy(
                ones_v, cnt_sh.at[dstv.at[r]], csem[bank][b], add=True)
        for b in range(2):
          i = t * SUP + q * 4 + 2 * bank + b
          r = lax.rem(i, 2 * SUP)
          pltpu.make_async_copy(
              rows[bank][b], acc_sh.at[dstv.at[r]], ssem[bank][b]).wait()
          if with_counts:
            pltpu.make_async_copy(
                ones_v, cnt_sh.at[dstv.at[r]], csem[bank][b]).wait()

          @pl.when(i + 4 < C)
          def _():
            rn = lax.rem(i + 4, 2 * SUP)
            pltpu.async_copy(
                table_h.at[srcv.at[rn]], rows[bank][b], gsem[bank][b])

    def superstep(t, carry):
      nxt = lax.rem((t + 1) * SUP, 2 * SUP)

      @pl.when(t + 1 < NSUP)
      def _():
        pltpu.async_copy(src_h.at[pl.ds(base + (t + 1) * SUP, SUP)],
                         srcv.at[pl.ds(nxt, SUP)], isem)
        pltpu.async_copy(dst_h.at[pl.ds(base + (t + 1) * SUP, SUP)],
                         dstv.at[pl.ds(nxt, SUP)], isem)

      def inner(q, carry2):
        quad(t, q)
        return carry2

      lax.fori_loop(0, 3, inner, 0)

      @pl.when(t + 1 < NSUP)
      def _():
        pltpu.make_async_copy(src_h.at[pl.ds(base + (t + 1) * SUP, SUP)],
                              srcv.at[pl.ds(nxt, SUP)], isem).wait()
        pltpu.make_async_copy(dst_h.at[pl.ds(base + (t + 1) * SUP, SUP)],
                              dstv.at[pl.ds(nxt, SUP)], isem).wait()

      quad(t, 3)
      return carry

    with jax.named_scope("edge_loop"):
      lax.fori_loop(0, NSUP, superstep, 0)

    plsc.subcore_barrier()

    with jax.named_scope("publish"):
      pltpu.sync_copy(
          acc_sh.at[pl.ds(s * ROWS_PER_TILE, ROWS_PER_TILE)],
          out_h.at[core, pl.ds(s * ROWS_PER_TILE, ROWS_PER_TILE)],
      )
      if with_counts:
        pltpu.sync_copy(
            cnt_sh.at[pl.ds(s * ROWS_PER_TILE, ROWS_PER_TILE)],
            cout_h.at[core, pl.ds(s * ROWS_PER_TILE, ROWS_PER_TILE)],
        )

  return sc_aggregate


BN = 2504


def _layer1_body(parts_ref, cnt_ref, x_ref, wl_ref, wr_ref, b_ref,
                 h_ref, inv_ref):
  cnt = cnt_ref[0, :, 0:1] + cnt_ref[1, :, 0:1]
  inv = 1.0 / jnp.maximum(cnt, 1.0)
  agg = (parts_ref[0] + parts_ref[1]) * inv
  h = jnp.dot(agg, wl_ref[...], preferred_element_type=jnp.float32)
  h += jnp.dot(x_ref[...], wr_ref[...], preferred_element_type=jnp.float32)
  h += b_ref[...]
  h_ref[...] = jnp.maximum(h, 0.0)
  inv_ref[...] = jnp.broadcast_to(inv, (BN, D_HID))


def _layer2_body(parts_ref, h_ref, inv_ref, wl_ref, wr_ref, b_ref, o_ref):
  agg = (parts_ref[0] + parts_ref[1]) * inv_ref[:, 0:1]
  o = jnp.dot(agg, wl_ref[...], preferred_element_type=jnp.float32)
  o += jnp.dot(h_ref[...], wr_ref[...], preferred_element_type=jnp.float32)
  o_ref[...] = o + b_ref[...]


def _tc_layer1(parts, cnts, x_pad, W1_l, W1_r, b1):
  grid = (NPAD // BN,)
  return pl.pallas_call(
      _layer1_body,
      grid=grid,
      in_specs=[
          pl.BlockSpec((NC, BN, D_FEAT), lambda i: (0, i, 0)),
          pl.BlockSpec((NC, BN, DC), lambda i: (0, i, 0)),
          pl.BlockSpec((BN, D_FEAT), lambda i: (i, 0)),
          pl.BlockSpec((D_FEAT, D_HID), lambda i: (0, 0)),
          pl.BlockSpec((D_FEAT, D_HID), lambda i: (0, 0)),
          pl.BlockSpec((1, D_HID), lambda i: (0, 0)),
      ],
      out_specs=[
          pl.BlockSpec((BN, D_HID), lambda i: (i, 0)),
          pl.BlockSpec((BN, D_HID), lambda i: (i, 0)),
      ],
      out_shape=[
          jax.ShapeDtypeStruct((NPAD, D_HID), jnp.float32),
          jax.ShapeDtypeStruct((NPAD, D_HID), jnp.float32),
      ],
  )(parts, cnts, x_pad, W1_l, W1_r, b1)


def _tc_layer2(parts, h, inv_b, W2_l, W2_r, b2):
  grid = (NPAD // BN,)
  return pl.pallas_call(
      _layer2_body,
      grid=grid,
      in_specs=[
          pl.BlockSpec((NC, BN, D_HID), lambda i: (0, i, 0)),
          pl.BlockSpec((BN, D_HID), lambda i: (i, 0)),
          pl.BlockSpec((BN, D_HID), lambda i: (i, 0)),
          pl.BlockSpec((D_HID, N_LABELS), lambda i: (0, 0)),
          pl.BlockSpec((D_HID, N_LABELS), lambda i: (0, 0)),
          pl.BlockSpec((1, N_LABELS), lambda i: (0, 0)),
      ],
      out_specs=pl.BlockSpec((BN, N_LABELS), lambda i: (i, 0)),
      out_shape=jax.ShapeDtypeStruct((NPAD, N_LABELS), jnp.float32),
  )(parts, h, inv_b, W2_l, W2_r, b2)


def kernel(x, edge_index, W1_l, W1_r, b1, W2_l, W2_r, b2):
  src = edge_index[0].astype(jnp.int32)
  dst = edge_index[1].astype(jnp.int32)
  sentinel = (N_NODES + jnp.arange(NW, dtype=jnp.int32)[:, None] // NC
              ) * jnp.ones((NW, PAD_PW), jnp.int32)
  src_p = jnp.concatenate(
      [src.reshape(NW, REAL_PW), sentinel], axis=1).reshape(NW * C, K)
  dst_p = jnp.concatenate(
      [dst.reshape(NW, REAL_PW), sentinel], axis=1).reshape(NW * C, K)

  x_pad = jnp.zeros((NPAD, D_FEAT), jnp.float32).at[:N_NODES].set(x)

  zeros_d = jnp.zeros((ROWS_PER_TILE, D_FEAT), jnp.float32)
  zeros_c = jnp.zeros((ROWS_PER_TILE, DC), jnp.float32)
  ones_b = jnp.ones((K, DC), jnp.float32)

  parts1, cnts = _make_sc_aggregate(D_FEAT, True)(
      src_p, dst_p, x_pad, zeros_d, zeros_c, ones_b)
  h, inv_b = _tc_layer1(parts1, cnts, x_pad, W1_l, W1_r,
                        b1.reshape(1, D_HID))
  (parts2,) = _make_sc_aggregate(D_HID, False)(src_p, dst_p, h, zeros_d)
  out = _tc_layer2(parts2, h, inv_b, W2_l, W2_r, b2.reshape(1, N_LABELS))
  return out[:N_NODES]
